# Initial kernel scaffold; baseline (speedup 1.0000x reference)
import jax
import jax.numpy as jnp
from jax.experimental import pallas as pl


def kernel(gu, gi, users, items, Tu_weight, F_feat, proj_W, proj_b):
    raise NotImplementedError("write your pallas kernel here")


# speedup vs baseline: 2.1535x; 2.1535x over previous
import functools

import jax
import jax.numpy as jnp
from jax import lax
from jax.experimental import pallas as pl
from jax.experimental.pallas import tpu as pltpu
from jax.experimental.pallas import tpu_sc as plsc

B = 16384
EMBED_K = 64
FEAT_DIM = 512

_NC = 2
_NS = 16
_NW = _NC * _NS
_BPW = B // _NW
_FCHUNK = 128
_NCHUNK = _BPW // _FCHUNK


def _sc_gather_f(items, F_feat):
    mesh = plsc.VectorSubcoreMesh(core_axis_name="c", subcore_axis_name="s")

    @functools.partial(
        pl.kernel,
        mesh=mesh,
        out_type=jax.ShapeDtypeStruct((B, FEAT_DIM), jnp.float32),
        scratch_types=[
            pltpu.VMEM((_FCHUNK,), jnp.int32),
            pltpu.VMEM((_FCHUNK, FEAT_DIM), jnp.float32),
            pltpu.SemaphoreType.DMA,
        ],
    )
    def k(items_hbm, f_hbm, effe_out, iidx_v, rows_v, sem):
        wid = lax.axis_index("s") * _NC + lax.axis_index("c")
        base = wid * _BPW
        for c in range(_NCHUNK):
            off = base + c * _FCHUNK
            pltpu.sync_copy(items_hbm.at[pl.ds(off, _FCHUNK)], iidx_v)
            pltpu.async_copy(f_hbm.at[iidx_v], rows_v, sem).wait()
            pltpu.sync_copy(rows_v, effe_out.at[pl.ds(off, _FCHUNK)])

    return k(items, F_feat)


def _sc_gather_tu(users, Tu_weight):
    mesh = plsc.VectorSubcoreMesh(core_axis_name="c", subcore_axis_name="s")

    @functools.partial(
        pl.kernel,
        mesh=mesh,
        out_type=jax.ShapeDtypeStruct((B, EMBED_K), jnp.float32),
        scratch_types=[
            pltpu.VMEM((_BPW,), jnp.int32),
            pltpu.VMEM((_BPW, EMBED_K), jnp.float32),
            pltpu.SemaphoreType.DMA,
        ],
        compiler_params=pltpu.CompilerParams(use_tc_tiling_on_sc=False),
    )
    def k(users_hbm, tu_hbm, theta_out, uidx_v, theta_v, sem):
        wid = lax.axis_index("s") * _NC + lax.axis_index("c")
        base = wid * _BPW
        pltpu.sync_copy(users_hbm.at[pl.ds(base, _BPW)], uidx_v)
        pltpu.async_copy(tu_hbm.at[uidx_v], theta_v, sem).wait()
        pltpu.sync_copy(theta_v, theta_out.at[pl.ds(base, _BPW)])

    return k(users, Tu_weight)


_TBLK = 1024


def _tc_dense_body(gu_ref, gi_ref, theta_ref, effe_ref, w_ref, b_ref,
                   xui_ref, proj_ref):
    proj = jnp.dot(effe_ref[...], w_ref[...],
                   preferred_element_type=jnp.float32) + b_ref[...]
    nrm = jnp.sqrt(jnp.sum(proj * proj, axis=1, keepdims=True))
    proj_i = proj / jnp.maximum(nrm, 1e-12)
    xui_ref[...] = (jnp.sum(gu_ref[...] * gi_ref[...], axis=1)
                    + jnp.sum(theta_ref[...] * proj_i, axis=1))
    proj_ref[...] = proj_i


def _tc_dense(gu, gi, theta_u, effe_i, proj_W, proj_b):
    return pl.pallas_call(
        _tc_dense_body,
        grid=(B // _TBLK,),
        in_specs=[
            pl.BlockSpec((_TBLK, EMBED_K), lambda i: (i, 0)),
            pl.BlockSpec((_TBLK, EMBED_K), lambda i: (i, 0)),
            pl.BlockSpec((_TBLK, EMBED_K), lambda i: (i, 0)),
            pl.BlockSpec((_TBLK, FEAT_DIM), lambda i: (i, 0)),
            pl.BlockSpec((FEAT_DIM, EMBED_K), lambda i: (0, 0)),
            pl.BlockSpec((1, EMBED_K), lambda i: (0, 0)),
        ],
        out_specs=[
            pl.BlockSpec((_TBLK,), lambda i: (i,)),
            pl.BlockSpec((_TBLK, EMBED_K), lambda i: (i, 0)),
        ],
        out_shape=[
            jax.ShapeDtypeStruct((B,), jnp.float32),
            jax.ShapeDtypeStruct((B, EMBED_K), jnp.float32),
        ],
    )(gu, gi, theta_u, effe_i, proj_W, proj_b.reshape(1, EMBED_K))


def kernel(gu, gi, users, items, Tu_weight, F_feat, proj_W, proj_b):
    theta_u = _sc_gather_tu(users, Tu_weight)
    effe_i = _sc_gather_f(items, F_feat)
    xui, proj_i = _tc_dense(gu, gi, theta_u, effe_i, proj_W, proj_b)
    return (xui, gu, gi, theta_u, proj_i)
